# baseline (speedup 1.0000x reference)
import functools

import jax
import jax.numpy as jnp
from jax.experimental import pallas as pl
from jax.experimental.pallas import tpu as pltpu

_K = 5
_C1, _C2, _C3 = 16, 32, 64


def _pow2_block(n, pref):
    b = pref
    while n % b:
        b //= 2
    return b


def _conv1_pool1(xt_ref, w1_ref, b1_ref, o_ref, s1, *, L1, P1):
    for c in range(_C1):
        fir = xt_ref[0:L1, :] * w1_ref[0:1, c:c + 1]
        for k in range(1, _K):
            fir = fir + xt_ref[k:k + L1, :] * w1_ref[k:k + 1, c:c + 1]
        s1[...] = jnp.maximum(fir + b1_ref[0:1, c:c + 1], 0.0)
        o_ref[c] = jnp.maximum(s1[pl.ds(0, P1, stride=2), :],
                               s1[pl.ds(1, P1, stride=2), :]).astype(jnp.bfloat16)


def _conv23(a_ref, w2_ref, b2_ref, w3_ref, b3_ref, feat_ref, p2, s2, p3, s3,
            *, SB, L2, L3, MPL3):
    bf16 = jnp.bfloat16
    for s in range(SB):
        for k in range(_K):
            p2[pl.ds(s * L2, L2), k * _C1:(k + 1) * _C1] = a_ref[s, k:k + L2, :]
    y2 = jnp.dot(p2[...], w2_ref[...], preferred_element_type=jnp.float32)
    s2[...] = jnp.maximum(y2 + b2_ref[...], 0.0)

    for s in range(SB):
        for k in range(_K):
            p3[pl.ds(s * L3, L3), k * _C2:(k + 1) * _C2] = jnp.maximum(
                s2[pl.ds(s * L2 + 2 * k, L3, stride=2), :],
                s2[pl.ds(s * L2 + 2 * k + 1, L3, stride=2), :]).astype(bf16)
    y3 = jnp.dot(p3[...], w3_ref[...], preferred_element_type=jnp.float32)
    s3[...] = jnp.maximum(y3 + b3_ref[...], 0.0)

    for s in range(SB):
        feat_ref[s] = jnp.maximum(
            s3[pl.ds(s * L3, MPL3, stride=2), :],
            s3[pl.ds(s * L3 + 1, MPL3, stride=2), :]).astype(bf16)


def _linear(a_ref, w_ref, b_ref, o_ref):
    o_ref[...] = jnp.dot(a_ref[...], w_ref[...],
                         preferred_element_type=jnp.float32) + b_ref[...]


def kernel(w1, b1, w2, b2, w3, b3, wl, bl, x):
    B, dim = x.shape
    L1 = dim - _K + 1
    P1 = L1 // 2
    L2 = P1 - _K + 1
    P2 = L2 // 2
    L3 = P2 - _K + 1
    MPL3 = L3 // 2
    F = _C3 * MPL3
    O = wl.shape[1]
    assert wl.shape[0] == F

    w2b = w2.astype(jnp.bfloat16)
    w3b = w3.astype(jnp.bfloat16)
    wlb = (wl.reshape(_C3, MPL3, O).transpose(1, 0, 2)
           .reshape(F, O).astype(jnp.bfloat16))
    b1r = b1.reshape(1, _C1)
    b2r = b2.reshape(1, _C2)
    b3r = b3.reshape(1, _C3)
    blr = bl.reshape(1, O)

    SL = _pow2_block(B, 128)
    xt = x.T
    a1 = pl.pallas_call(
        functools.partial(_conv1_pool1, L1=L1, P1=P1),
        out_shape=jax.ShapeDtypeStruct((_C1, P1, B), jnp.bfloat16),
        grid=(B // SL,),
        in_specs=[
            pl.BlockSpec((dim, SL), lambda g: (0, g)),
            pl.BlockSpec((_K, _C1), lambda g: (0, 0)),
            pl.BlockSpec((1, _C1), lambda g: (0, 0)),
        ],
        out_specs=pl.BlockSpec((_C1, P1, SL), lambda g: (0, 0, g)),
        scratch_shapes=[pltpu.VMEM((L1, SL), jnp.float32)],
        compiler_params=pltpu.CompilerParams(
            dimension_semantics=("parallel",)),
    )(xt, w1, b1r)

    a1t = jnp.transpose(a1, (2, 1, 0))

    SB = _pow2_block(B, 16)
    feats = pl.pallas_call(
        functools.partial(_conv23, SB=SB, L2=L2, L3=L3, MPL3=MPL3),
        out_shape=jax.ShapeDtypeStruct((B, MPL3, _C3), jnp.bfloat16),
        grid=(B // SB,),
        in_specs=[
            pl.BlockSpec((SB, P1, _C1), lambda g: (g, 0, 0)),
            pl.BlockSpec((_K * _C1, _C2), lambda g: (0, 0)),
            pl.BlockSpec((1, _C2), lambda g: (0, 0)),
            pl.BlockSpec((_K * _C2, _C3), lambda g: (0, 0)),
            pl.BlockSpec((1, _C3), lambda g: (0, 0)),
        ],
        out_specs=pl.BlockSpec((SB, MPL3, _C3), lambda g: (g, 0, 0)),
        scratch_shapes=[
            pltpu.VMEM((SB * L2, _K * _C1), jnp.bfloat16),
            pltpu.VMEM((SB * L2, _C2), jnp.float32),
            pltpu.VMEM((SB * L3, _K * _C2), jnp.bfloat16),
            pltpu.VMEM((SB * L3, _C3), jnp.float32),
        ],
        compiler_params=pltpu.CompilerParams(
            dimension_semantics=("parallel",)),
    )(a1t, w2b, b2r, w3b, b3r)

    flat = feats.reshape(B, F)
    BM = _pow2_block(B, 256)
    out = pl.pallas_call(
        _linear,
        out_shape=jax.ShapeDtypeStruct((B, O), jnp.float32),
        grid=(B // BM,),
        in_specs=[
            pl.BlockSpec((BM, F), lambda i: (i, 0)),
            pl.BlockSpec((F, O), lambda i: (0, 0)),
            pl.BlockSpec((1, O), lambda i: (0, 0)),
        ],
        out_specs=pl.BlockSpec((BM, O), lambda i: (i, 0)),
        compiler_params=pltpu.CompilerParams(
            dimension_semantics=("parallel",)),
    )(flat, wlb, blr)
    return out


# speedup vs baseline: 1.0581x; 1.0581x over previous
import functools

import jax
import jax.numpy as jnp
from jax.experimental import pallas as pl
from jax.experimental.pallas import tpu as pltpu

_K = 5
_C1, _C2, _C3 = 16, 32, 64


def _pow2_block(n, pref):
    b = pref
    while n % b:
        b //= 2
    return b


def _conv1_pool1(xt_ref, w1_ref, b1_ref, o_ref, s1, *, L1, P1):
    for c in range(_C1):
        fir = xt_ref[0:L1, :] * w1_ref[0:1, c:c + 1]
        for k in range(1, _K):
            fir = fir + xt_ref[k:k + L1, :] * w1_ref[k:k + 1, c:c + 1]
        s1[...] = jnp.maximum(fir + b1_ref[0:1, c:c + 1], 0.0)
        o_ref[c] = jnp.maximum(s1[pl.ds(0, P1, stride=2), :],
                               s1[pl.ds(1, P1, stride=2), :]).astype(jnp.bfloat16)


def _conv23(a_ref, w2_ref, b2_ref, w3_ref, b3_ref, feat_ref, p2, s2, p3, s3,
            *, SB, L2, L3, MPL3):
    bf16 = jnp.bfloat16
    for s in range(SB):
        for k in range(_K):
            p2[pl.ds(s * L2, L2), k * _C1:(k + 1) * _C1] = a_ref[s, k:k + L2, :]
    y2 = jnp.dot(p2[...], w2_ref[...], preferred_element_type=jnp.float32)
    s2[...] = jnp.maximum(y2 + b2_ref[...], 0.0)

    for s in range(SB):
        for k in range(_K):
            p3[pl.ds(s * L3, L3), k * _C2:(k + 1) * _C2] = jnp.maximum(
                s2[pl.ds(s * L2 + 2 * k, L3, stride=2), :],
                s2[pl.ds(s * L2 + 2 * k + 1, L3, stride=2), :]).astype(bf16)
    y3 = jnp.dot(p3[...], w3_ref[...], preferred_element_type=jnp.float32)
    s3[...] = jnp.maximum(y3 + b3_ref[...], 0.0)

    for s in range(SB):
        feat_ref[s] = jnp.maximum(
            s3[pl.ds(s * L3, MPL3, stride=2), :],
            s3[pl.ds(s * L3 + 1, MPL3, stride=2), :]).astype(bf16)


def _linear(a_ref, w_ref, b_ref, o_ref):
    o_ref[...] = jnp.dot(a_ref[...], w_ref[...],
                         preferred_element_type=jnp.float32) + b_ref[...]


def kernel(w1, b1, w2, b2, w3, b3, wl, bl, x):
    B, dim = x.shape
    L1 = dim - _K + 1
    P1 = L1 // 2
    L2 = P1 - _K + 1
    P2 = L2 // 2
    L3 = P2 - _K + 1
    MPL3 = L3 // 2
    F = _C3 * MPL3
    O = wl.shape[1]
    assert wl.shape[0] == F

    w2b = w2.astype(jnp.bfloat16)
    w3b = w3.astype(jnp.bfloat16)
    wlb = (wl.reshape(_C3, MPL3, O).transpose(1, 0, 2)
           .reshape(F, O).astype(jnp.bfloat16))
    b1r = b1.reshape(1, _C1)
    b2r = b2.reshape(1, _C2)
    b3r = b3.reshape(1, _C3)
    blr = bl.reshape(1, O)

    SL = _pow2_block(B, 128)
    xt = x.T
    a1 = pl.pallas_call(
        functools.partial(_conv1_pool1, L1=L1, P1=P1),
        out_shape=jax.ShapeDtypeStruct((_C1, P1, B), jnp.bfloat16),
        grid=(B // SL,),
        in_specs=[
            pl.BlockSpec((dim, SL), lambda g: (0, g)),
            pl.BlockSpec((_K, _C1), lambda g: (0, 0)),
            pl.BlockSpec((1, _C1), lambda g: (0, 0)),
        ],
        out_specs=pl.BlockSpec((_C1, P1, SL), lambda g: (0, 0, g)),
        scratch_shapes=[pltpu.VMEM((L1, SL), jnp.float32)],
        compiler_params=pltpu.CompilerParams(
            dimension_semantics=("parallel",)),
    )(xt, w1, b1r)

    a1t = jnp.transpose(a1, (2, 1, 0))

    SB = _pow2_block(B, 8)
    feats = pl.pallas_call(
        functools.partial(_conv23, SB=SB, L2=L2, L3=L3, MPL3=MPL3),
        out_shape=jax.ShapeDtypeStruct((B, MPL3, _C3), jnp.bfloat16),
        grid=(B // SB,),
        in_specs=[
            pl.BlockSpec((SB, P1, _C1), lambda g: (g, 0, 0)),
            pl.BlockSpec((_K * _C1, _C2), lambda g: (0, 0)),
            pl.BlockSpec((1, _C2), lambda g: (0, 0)),
            pl.BlockSpec((_K * _C2, _C3), lambda g: (0, 0)),
            pl.BlockSpec((1, _C3), lambda g: (0, 0)),
        ],
        out_specs=pl.BlockSpec((SB, MPL3, _C3), lambda g: (g, 0, 0)),
        scratch_shapes=[
            pltpu.VMEM((SB * L2, _K * _C1), jnp.bfloat16),
            pltpu.VMEM((SB * L2, _C2), jnp.float32),
            pltpu.VMEM((SB * L3, _K * _C2), jnp.bfloat16),
            pltpu.VMEM((SB * L3, _C3), jnp.float32),
        ],
        compiler_params=pltpu.CompilerParams(
            dimension_semantics=("parallel",)),
    )(a1t, w2b, b2r, w3b, b3r)

    flat = feats.reshape(B, F)
    BM = _pow2_block(B, 256)
    out = pl.pallas_call(
        _linear,
        out_shape=jax.ShapeDtypeStruct((B, O), jnp.float32),
        grid=(B // BM,),
        in_specs=[
            pl.BlockSpec((BM, F), lambda i: (i, 0)),
            pl.BlockSpec((F, O), lambda i: (0, 0)),
            pl.BlockSpec((1, O), lambda i: (0, 0)),
        ],
        out_specs=pl.BlockSpec((BM, O), lambda i: (i, 0)),
        compiler_params=pltpu.CompilerParams(
            dimension_semantics=("parallel",)),
    )(flat, wlb, blr)
    return out


# speedup vs baseline: 1.0994x; 1.0390x over previous
import functools

import jax
import jax.numpy as jnp
from jax.experimental import pallas as pl
from jax.experimental.pallas import tpu as pltpu

_K = 5
_C1, _C2, _C3 = 16, 32, 64


def _pow2_block(n, pref):
    b = pref
    while n % b:
        b //= 2
    return b


def _conv1_pool1(xt_ref, w1_ref, b1_ref, o_ref, s1, *, L1, P1):
    for c in range(_C1):
        fir = xt_ref[0:L1, :] * w1_ref[0:1, c:c + 1]
        for k in range(1, _K):
            fir = fir + xt_ref[k:k + L1, :] * w1_ref[k:k + 1, c:c + 1]
        s1[...] = jnp.maximum(fir + b1_ref[0:1, c:c + 1], 0.0)
        o_ref[c] = jnp.maximum(s1[pl.ds(0, P1, stride=2), :],
                               s1[pl.ds(1, P1, stride=2), :]).astype(jnp.bfloat16)


def _conv23(a_ref, w2_ref, b2_ref, w3_ref, b3_ref, feat_ref, p2, s2, p3, s3,
            *, SB, L2, L3, MPL3):
    bf16 = jnp.bfloat16
    for s in range(SB):
        for k in range(_K):
            p2[pl.ds(s * L2, L2), k * _C1:(k + 1) * _C1] = a_ref[s, k:k + L2, :]
    y2 = jnp.dot(p2[...], w2_ref[...], preferred_element_type=jnp.float32)
    s2[...] = jnp.maximum(y2 + b2_ref[...], 0.0)

    for s in range(SB):
        for k in range(_K):
            p3[pl.ds(s * L3, L3), k * _C2:(k + 1) * _C2] = jnp.maximum(
                s2[pl.ds(s * L2 + 2 * k, L3, stride=2), :],
                s2[pl.ds(s * L2 + 2 * k + 1, L3, stride=2), :]).astype(bf16)
    y3 = jnp.dot(p3[...], w3_ref[...], preferred_element_type=jnp.float32)
    s3[...] = jnp.maximum(y3 + b3_ref[...], 0.0)

    for s in range(SB):
        feat_ref[s] = jnp.maximum(
            s3[pl.ds(s * L3, MPL3, stride=2), :],
            s3[pl.ds(s * L3 + 1, MPL3, stride=2), :]).astype(bf16)


def _linear(a_ref, w_ref, b_ref, o_ref):
    o_ref[...] = jnp.dot(a_ref[...], w_ref[...],
                         preferred_element_type=jnp.float32) + b_ref[...]


def kernel(w1, b1, w2, b2, w3, b3, wl, bl, x):
    B, dim = x.shape
    L1 = dim - _K + 1
    P1 = L1 // 2
    L2 = P1 - _K + 1
    P2 = L2 // 2
    L3 = P2 - _K + 1
    MPL3 = L3 // 2
    F = _C3 * MPL3
    O = wl.shape[1]
    assert wl.shape[0] == F

    w2b = w2.astype(jnp.bfloat16)
    w3b = w3.astype(jnp.bfloat16)
    wlb = (wl.reshape(_C3, MPL3, O).transpose(1, 0, 2)
           .reshape(F, O).astype(jnp.bfloat16))
    b1r = b1.reshape(1, _C1)
    b2r = b2.reshape(1, _C2)
    b3r = b3.reshape(1, _C3)
    blr = bl.reshape(1, O)

    SL = _pow2_block(B, 128)
    xt = x.T
    a1 = pl.pallas_call(
        functools.partial(_conv1_pool1, L1=L1, P1=P1),
        out_shape=jax.ShapeDtypeStruct((_C1, P1, B), jnp.bfloat16),
        grid=(B // SL,),
        in_specs=[
            pl.BlockSpec((dim, SL), lambda g: (0, g)),
            pl.BlockSpec((_K, _C1), lambda g: (0, 0)),
            pl.BlockSpec((1, _C1), lambda g: (0, 0)),
        ],
        out_specs=pl.BlockSpec((_C1, P1, SL), lambda g: (0, 0, g)),
        scratch_shapes=[pltpu.VMEM((L1, SL), jnp.float32)],
        compiler_params=pltpu.CompilerParams(
            dimension_semantics=("parallel",)),
    )(xt, w1, b1r)

    a1t = (a1.reshape(_C1 * P1, B).T
           .reshape(B, _C1, P1).swapaxes(1, 2))

    SB = _pow2_block(B, 8)
    feats = pl.pallas_call(
        functools.partial(_conv23, SB=SB, L2=L2, L3=L3, MPL3=MPL3),
        out_shape=jax.ShapeDtypeStruct((B, MPL3, _C3), jnp.bfloat16),
        grid=(B // SB,),
        in_specs=[
            pl.BlockSpec((SB, P1, _C1), lambda g: (g, 0, 0)),
            pl.BlockSpec((_K * _C1, _C2), lambda g: (0, 0)),
            pl.BlockSpec((1, _C2), lambda g: (0, 0)),
            pl.BlockSpec((_K * _C2, _C3), lambda g: (0, 0)),
            pl.BlockSpec((1, _C3), lambda g: (0, 0)),
        ],
        out_specs=pl.BlockSpec((SB, MPL3, _C3), lambda g: (g, 0, 0)),
        scratch_shapes=[
            pltpu.VMEM((SB * L2, _K * _C1), jnp.bfloat16),
            pltpu.VMEM((SB * L2, _C2), jnp.float32),
            pltpu.VMEM((SB * L3, _K * _C2), jnp.bfloat16),
            pltpu.VMEM((SB * L3, _C3), jnp.float32),
        ],
        compiler_params=pltpu.CompilerParams(
            dimension_semantics=("parallel",)),
    )(a1t, w2b, b2r, w3b, b3r)

    flat = feats.reshape(B, F)
    BM = _pow2_block(B, 256)
    out = pl.pallas_call(
        _linear,
        out_shape=jax.ShapeDtypeStruct((B, O), jnp.float32),
        grid=(B // BM,),
        in_specs=[
            pl.BlockSpec((BM, F), lambda i: (i, 0)),
            pl.BlockSpec((F, O), lambda i: (0, 0)),
            pl.BlockSpec((1, O), lambda i: (0, 0)),
        ],
        out_specs=pl.BlockSpec((BM, O), lambda i: (i, 0)),
        compiler_params=pltpu.CompilerParams(
            dimension_semantics=("parallel",)),
    )(flat, wlb, blr)
    return out
